# baseline (speedup 1.0000x reference)
import functools

import jax
import jax.numpy as jnp
from jax import lax
from jax.experimental import pallas as pl
from jax.experimental.pallas import tpu as pltpu
from jax.experimental.pallas import tpu_sc as plsc

_L = 16
_NUM_TYPES = 64
_ATT_DIM = 256
_KB = _NUM_TYPES // _L


_ROWS_PER_CHUNK = 32


@functools.partial(jax.jit, static_argnames=("sc_rows",))
def _dep_type_sc(embT_flat, wq, adj2d, sc_rows):
    info = plsc.get_sparse_core_info()
    nc, ns = info.num_cores, info.num_subcores
    nw = nc * ns
    nrows, ncols = adj2d.shape
    assert ncols % _L == 0 and sc_rows % nw == 0
    rows_per_w = sc_rows // nw
    rch = _ROWS_PER_CHUNK
    vecs_per_row = ncols // _L
    assert rows_per_w % rch == 0
    n_chunks = rows_per_w // rch
    assert n_chunks % 2 == 0 and n_chunks >= 4

    mesh = plsc.VectorSubcoreMesh(core_axis_name="c", subcore_axis_name="s")

    @functools.partial(
        pl.kernel,
        mesh=mesh,
        out_type=jax.ShapeDtypeStruct((sc_rows, ncols), jnp.float32),
        compiler_params=pltpu.CompilerParams(needs_layout_passes=False),
        scratch_types=[
            pltpu.VMEM((_ATT_DIM * _NUM_TYPES,), jnp.float32),
            pltpu.VMEM((_ATT_DIM,), jnp.float32),
            pltpu.VMEM((_NUM_TYPES,), jnp.float32),
            pltpu.VMEM((_NUM_TYPES * _L,), jnp.float32),
            pltpu.VMEM((rch, ncols), jnp.int32),
            pltpu.VMEM((rch, ncols), jnp.int32),
            pltpu.VMEM((rch, ncols), jnp.float32),
            pltpu.VMEM((rch, ncols), jnp.float32),
            pltpu.VMEM((_L,), jnp.float32),
            pltpu.SemaphoreType.DMA,
            pltpu.SemaphoreType.DMA,
            pltpu.SemaphoreType.DMA,
            pltpu.SemaphoreType.DMA,
        ],
    )
    def k(embT_hbm, wq_hbm, adj_hbm, out_hbm, embT_v, wq_v, table_v, rep_v,
          idx0_v, idx1_v, res0_v, res1_v, red_v, ins0, ins1, outs0, outs1):
        wid = lax.axis_index("s") * nc + lax.axis_index("c")

        pltpu.sync_copy(embT_hbm, embT_v)
        pltpu.sync_copy(wq_hbm, wq_v)

        def matvec_body(db, accs):
            wv = wq_v[pl.ds(db * _L, _L)]
            accs = list(accs)
            for j in range(_L):
                d = db * _L + j
                w = wv[j]
                for kb in range(_KB):
                    accs[kb] = accs[kb] + embT_v[pl.ds(d * _NUM_TYPES + kb * _L, _L)] * w
            return tuple(accs)

        zero = jnp.zeros((_L,), jnp.float32)
        q = lax.fori_loop(0, _ATT_DIM // _L, matvec_body, (zero,) * _KB)

        lanes = lax.iota(jnp.int32, _L)

        def xlane_reduce(v, op):
            for k_sh in (8, 4, 2, 1):
                red_v[pl.ds(0, _L)] = v
                v = op(v, plsc.load_gather(red_v, [lanes ^ k_sh]))
            return v

        m = xlane_reduce(
            jnp.maximum(jnp.maximum(q[0], q[1]), jnp.maximum(q[2], q[3])),
            jnp.maximum,
        )
        e = tuple(jnp.exp(q[kb] - m) for kb in range(_KB))
        s = xlane_reduce(e[0] + e[1] + e[2] + e[3], jnp.add)
        t0 = jnp.where(lanes == 0, jnp.float32(0.0), e[0] / s)
        table_v[pl.ds(0, _L)] = t0
        for kb in range(1, _KB):
            table_v[pl.ds(kb * _L, _L)] = e[kb] / s

        def rep_body(j, _):
            jv = lax.broadcast_in_dim(j, (_L,), ())
            rep_v[pl.ds(j * _L, _L)] = plsc.load_gather(table_v, [jv])
            return 0

        lax.fori_loop(0, _NUM_TYPES, rep_body, 0)

        base = wid * rows_per_w
        idx_bufs = (idx0_v, idx1_v)
        res_bufs = (res0_v, res1_v)
        in_sems = (ins0, ins1)
        out_sems = (outs0, outs1)

        def in_copy(c, b):
            return pltpu.make_async_copy(
                adj_hbm.at[pl.ds(base + c * rch, rch)], idx_bufs[b], in_sems[b])

        def out_copy(c, b):
            return pltpu.make_async_copy(
                res_bufs[b], out_hbm.at[pl.ds(base + c * rch, rch)], out_sems[b])

        in_copy(0, 0).start()
        in_copy(1, 1).start()

        def outer(half, _):
            for b in range(2):
                c = half * 2 + b
                in_copy(c, b).wait()

                @pl.when(c >= 2)
                def _():
                    out_copy(c - 2, b).wait()

                @plsc.parallel_loop(0, rch * vecs_per_row, unroll=8)
                def inner(i):
                    r = i >> 5
                    col = (i & (vecs_per_row - 1)) * _L
                    idx = idx_bufs[b][r, pl.ds(col, _L)]
                    gidx = (idx << 4) + lanes
                    res_bufs[b][r, pl.ds(col, _L)] = plsc.load_gather(
                        rep_v, [gidx])

                out_copy(c, b).start()

                @pl.when(c + 2 < n_chunks)
                def _():
                    in_copy(c + 2, b).start()

            return 0

        lax.fori_loop(0, n_chunks // 2, outer, 0)
        out_copy(n_chunks - 2, 0).wait()
        out_copy(n_chunks - 1, 1).wait()

    return k(embT_flat, wq, adj2d)


_TC_BR = 512


@functools.partial(jax.jit, static_argnames=("row0",))
def _dep_type_tc(emb, wq2, adj2d, row0):
    nrows, ncols = adj2d.shape
    f = nrows - row0
    assert f % _TC_BR == 0 and row0 % _TC_BR == 0
    lanes128 = 2 * _NUM_TYPES

    def body(emb_ref, wq_ref, idx_ref, out_ref, tab_ref):
        @pl.when(pl.program_id(0) == 0)
        def _():
            qv = lax.dot_general(
                wq_ref[...], emb_ref[...], (((1,), (1,)), ((), ())),
                preferred_element_type=jnp.float32)
            m = jnp.max(qv)
            ev = jnp.exp(qv - m)
            t = ev / jnp.sum(ev)
            t128 = jnp.concatenate(
                [t, jnp.zeros((1, _NUM_TYPES), jnp.float32)], axis=1)
            ln = lax.broadcasted_iota(jnp.int32, (1, lanes128), 1)
            tab_ref[...] = jnp.where(ln == 0, jnp.float32(0.0), t128)

        src = jnp.broadcast_to(tab_ref[...], (_TC_BR, lanes128))
        idx = idx_ref[...]
        out_ref[...] = jnp.concatenate(
            [jnp.take_along_axis(src, idx[:, j * 128:(j + 1) * 128], axis=1)
             for j in range(ncols // 128)], axis=1)

    return pl.pallas_call(
        body,
        grid=(f // _TC_BR,),
        in_specs=[
            pl.BlockSpec((_NUM_TYPES, _ATT_DIM), lambda i: (0, 0)),
            pl.BlockSpec((1, _ATT_DIM), lambda i: (0, 0)),
            pl.BlockSpec((_TC_BR, ncols), lambda i: (row0 // _TC_BR + i, 0)),
        ],
        out_specs=pl.BlockSpec((_TC_BR, ncols), lambda i: (i, 0)),
        out_shape=jax.ShapeDtypeStruct((f, ncols), jnp.float32),
        scratch_shapes=[pltpu.VMEM((1, lanes128), jnp.float32)],
    )(emb, wq2, adj2d)


def kernel(dep_emb_weight, syn_dep_adj, overall_max_len, batch_size, W_q, b_q):
    del overall_max_len, batch_size, b_q
    b, lq, lk = syn_dep_adj.shape
    embT = dep_emb_weight.T.reshape(-1)
    wq = W_q.reshape(-1)
    adj2d = syn_dep_adj.reshape(b * lq, lk)
    sc_rows = b * lq // 2
    out_sc = _dep_type_sc(embT, wq, adj2d, sc_rows)
    out_tc = _dep_type_tc(dep_emb_weight, W_q, adj2d, sc_rows)
    out = jnp.concatenate([out_sc, out_tc], axis=0)
    return out.reshape(b, lq, lk)


# speedup vs baseline: 1.2919x; 1.2919x over previous
import functools

import jax
import jax.numpy as jnp
from jax import lax
from jax.experimental import pallas as pl
from jax.experimental.pallas import tpu as pltpu
from jax.experimental.pallas import tpu_sc as plsc

_L = 16
_NUM_TYPES = 64
_ATT_DIM = 256
_KB = _NUM_TYPES // _L


_ROWS_PER_CHUNK = 32


@functools.partial(jax.jit, static_argnames=())
def _dep_type_sc(embT_flat, wq, adj2d):
    info = plsc.get_sparse_core_info()
    nc, ns = info.num_cores, info.num_subcores
    nw = nc * ns
    nrows, ncols = adj2d.shape
    assert ncols % _L == 0 and nrows % nw == 0
    rows_per_w = nrows // nw
    rch = _ROWS_PER_CHUNK
    vecs_per_row = ncols // _L
    assert rows_per_w % rch == 0
    n_chunks = rows_per_w // rch
    assert n_chunks % 2 == 0 and n_chunks >= 4

    mesh = plsc.VectorSubcoreMesh(core_axis_name="c", subcore_axis_name="s")

    @functools.partial(
        pl.kernel,
        mesh=mesh,
        out_type=jax.ShapeDtypeStruct((nrows, ncols), jnp.float32),
        compiler_params=pltpu.CompilerParams(needs_layout_passes=False),
        scratch_types=[
            pltpu.VMEM((_ATT_DIM * _NUM_TYPES,), jnp.float32),
            pltpu.VMEM((_ATT_DIM,), jnp.float32),
            pltpu.VMEM((_NUM_TYPES,), jnp.float32),
            pltpu.VMEM((_NUM_TYPES * _L,), jnp.float32),
            pltpu.VMEM((rch, ncols), jnp.int32),
            pltpu.VMEM((rch, ncols), jnp.int32),
            pltpu.VMEM((rch, ncols), jnp.float32),
            pltpu.VMEM((rch, ncols), jnp.float32),
            pltpu.VMEM((_L,), jnp.float32),
            pltpu.SemaphoreType.DMA,
            pltpu.SemaphoreType.DMA,
            pltpu.SemaphoreType.DMA,
            pltpu.SemaphoreType.DMA,
        ],
    )
    def k(embT_hbm, wq_hbm, adj_hbm, out_hbm, embT_v, wq_v, table_v, rep_v,
          idx0_v, idx1_v, res0_v, res1_v, red_v, ins0, ins1, outs0, outs1):
        wid = lax.axis_index("s") * nc + lax.axis_index("c")
        base = wid * rows_per_w

        def in_copy(c, b):
            return pltpu.make_async_copy(
                adj_hbm.at[pl.ds(base + c * rch, rch)],
                (idx0_v, idx1_v)[b], (ins0, ins1)[b])

        def out_copy(c, b):
            return pltpu.make_async_copy(
                (res0_v, res1_v)[b],
                out_hbm.at[pl.ds(base + c * rch, rch)], (outs0, outs1)[b])

        in_copy(0, 0).start()
        in_copy(1, 1).start()

        pltpu.sync_copy(embT_hbm, embT_v)
        pltpu.sync_copy(wq_hbm, wq_v)

        def matvec_body(db, accs):
            wv = wq_v[pl.ds(db * _L, _L)]
            accs = list(accs)
            for j in range(_L):
                d = db * _L + j
                w = wv[j]
                for kb in range(_KB):
                    accs[kb] = accs[kb] + embT_v[pl.ds(d * _NUM_TYPES + kb * _L, _L)] * w
            return tuple(accs)

        zero = jnp.zeros((_L,), jnp.float32)
        q = lax.fori_loop(0, _ATT_DIM // _L, matvec_body, (zero,) * _KB)

        lanes = lax.iota(jnp.int32, _L)

        def xlane_reduce(v, op):
            for k_sh in (8, 4, 2, 1):
                red_v[pl.ds(0, _L)] = v
                v = op(v, plsc.load_gather(red_v, [lanes ^ k_sh]))
            return v

        m = xlane_reduce(
            jnp.maximum(jnp.maximum(q[0], q[1]), jnp.maximum(q[2], q[3])),
            jnp.maximum,
        )
        e = tuple(jnp.exp(q[kb] - m) for kb in range(_KB))
        s = xlane_reduce(e[0] + e[1] + e[2] + e[3], jnp.add)
        t0 = jnp.where(lanes == 0, jnp.float32(0.0), e[0] / s)
        table_v[pl.ds(0, _L)] = t0
        for kb in range(1, _KB):
            table_v[pl.ds(kb * _L, _L)] = e[kb] / s

        def rep_body(j, _):
            jv = lax.broadcast_in_dim(j, (_L,), ())
            rep_v[pl.ds(j * _L, _L)] = plsc.load_gather(table_v, [jv])
            return 0

        lax.fori_loop(0, _NUM_TYPES, rep_body, 0)

        idx_bufs = (idx0_v, idx1_v)
        res_bufs = (res0_v, res1_v)

        def outer(half, _):
            for b in range(2):
                c = half * 2 + b
                in_copy(c, b).wait()

                @pl.when(c >= 2)
                def _():
                    out_copy(c - 2, b).wait()

                @plsc.parallel_loop(0, rch * vecs_per_row, unroll=8)
                def inner(i):
                    r = i >> 5
                    col = (i & (vecs_per_row - 1)) * _L
                    idx = idx_bufs[b][r, pl.ds(col, _L)]
                    gidx = (idx << 4) + lanes
                    res_bufs[b][r, pl.ds(col, _L)] = plsc.load_gather(
                        rep_v, [gidx])

                out_copy(c, b).start()

                @pl.when(c + 2 < n_chunks)
                def _():
                    in_copy(c + 2, b).start()

            return 0

        lax.fori_loop(0, n_chunks // 2, outer, 0)
        out_copy(n_chunks - 2, 0).wait()
        out_copy(n_chunks - 1, 1).wait()

    return k(embT_flat, wq, adj2d)


def kernel(dep_emb_weight, syn_dep_adj, overall_max_len, batch_size, W_q, b_q):
    del overall_max_len, batch_size, b_q
    b, lq, lk = syn_dep_adj.shape
    embT = dep_emb_weight.T.reshape(-1)
    wq = W_q.reshape(-1)
    adj2d = syn_dep_adj.reshape(b * lq, lk)
    out = _dep_type_sc(embT, wq, adj2d)
    return out.reshape(b, lq, lk)


# speedup vs baseline: 1.3581x; 1.0512x over previous
import functools

import jax
import jax.numpy as jnp
from jax import lax
from jax.experimental import pallas as pl
from jax.experimental.pallas import tpu as pltpu
from jax.experimental.pallas import tpu_sc as plsc

_L = 16
_NUM_TYPES = 64
_ATT_DIM = 256
_KB = _NUM_TYPES // _L


_ROWS_PER_CHUNK = 16
_NBUF = 4


@functools.partial(jax.jit, static_argnames=())
def _dep_type_sc(embT_flat, wq, adj2d):
    info = plsc.get_sparse_core_info()
    nc, ns = info.num_cores, info.num_subcores
    nw = nc * ns
    nrows, ncols = adj2d.shape
    assert ncols % _L == 0 and nrows % nw == 0
    rows_per_w = nrows // nw
    rch = _ROWS_PER_CHUNK
    vecs_per_row = ncols // _L
    assert rows_per_w % rch == 0
    n_chunks = rows_per_w // rch
    assert n_chunks % _NBUF == 0 and n_chunks >= 2 * _NBUF

    mesh = plsc.VectorSubcoreMesh(core_axis_name="c", subcore_axis_name="s")

    @functools.partial(
        pl.kernel,
        mesh=mesh,
        out_type=jax.ShapeDtypeStruct((nrows, ncols), jnp.float32),
        compiler_params=pltpu.CompilerParams(needs_layout_passes=False),
        scratch_types=[
            pltpu.VMEM((_ATT_DIM * _NUM_TYPES,), jnp.float32),
            pltpu.VMEM((_ATT_DIM,), jnp.float32),
            pltpu.VMEM((_NUM_TYPES,), jnp.float32),
            pltpu.VMEM((_NUM_TYPES * _L,), jnp.float32),
            *[pltpu.VMEM((rch, ncols), jnp.int32) for _ in range(_NBUF)],
            *[pltpu.VMEM((rch, ncols), jnp.float32) for _ in range(_NBUF)],
            pltpu.VMEM((_L,), jnp.float32),
            *[pltpu.SemaphoreType.DMA for _ in range(2 * _NBUF)],
        ],
    )
    def k(embT_hbm, wq_hbm, adj_hbm, out_hbm, embT_v, wq_v, table_v, rep_v,
          *bufs_and_sems):
        idx_bufs = bufs_and_sems[:_NBUF]
        res_bufs = bufs_and_sems[_NBUF:2 * _NBUF]
        red_v = bufs_and_sems[2 * _NBUF]
        in_sems = bufs_and_sems[2 * _NBUF + 1:2 * _NBUF + 1 + _NBUF]
        out_sems = bufs_and_sems[2 * _NBUF + 1 + _NBUF:]
        wid = lax.axis_index("s") * nc + lax.axis_index("c")
        base = wid * rows_per_w

        def in_copy(c, b):
            return pltpu.make_async_copy(
                adj_hbm.at[pl.ds(base + c * rch, rch)],
                idx_bufs[b], in_sems[b])

        def out_copy(c, b):
            return pltpu.make_async_copy(
                res_bufs[b],
                out_hbm.at[pl.ds(base + c * rch, rch)], out_sems[b])

        for b0 in range(_NBUF):
            in_copy(b0, b0).start()

        pltpu.sync_copy(embT_hbm, embT_v)
        pltpu.sync_copy(wq_hbm, wq_v)

        def matvec_body(db, accs):
            wv = wq_v[pl.ds(db * _L, _L)]
            accs = list(accs)
            for j in range(_L):
                d = db * _L + j
                w = wv[j]
                for kb in range(_KB):
                    accs[kb] = accs[kb] + embT_v[pl.ds(d * _NUM_TYPES + kb * _L, _L)] * w
            return tuple(accs)

        zero = jnp.zeros((_L,), jnp.float32)
        q = lax.fori_loop(0, _ATT_DIM // _L, matvec_body, (zero,) * _KB)

        lanes = lax.iota(jnp.int32, _L)

        def xlane_reduce(v, op):
            for k_sh in (8, 4, 2, 1):
                red_v[pl.ds(0, _L)] = v
                v = op(v, plsc.load_gather(red_v, [lanes ^ k_sh]))
            return v

        m = xlane_reduce(
            jnp.maximum(jnp.maximum(q[0], q[1]), jnp.maximum(q[2], q[3])),
            jnp.maximum,
        )
        e = tuple(jnp.exp(q[kb] - m) for kb in range(_KB))
        s = xlane_reduce(e[0] + e[1] + e[2] + e[3], jnp.add)
        t0 = jnp.where(lanes == 0, jnp.float32(0.0), e[0] / s)
        table_v[pl.ds(0, _L)] = t0
        for kb in range(1, _KB):
            table_v[pl.ds(kb * _L, _L)] = e[kb] / s

        def rep_body(j, _):
            jv = lax.broadcast_in_dim(j, (_L,), ())
            rep_v[pl.ds(j * _L, _L)] = plsc.load_gather(table_v, [jv])
            return 0

        lax.fori_loop(0, _NUM_TYPES, rep_body, 0)

        def outer(grp, _):
            for b in range(_NBUF):
                c = grp * _NBUF + b
                in_copy(c, b).wait()

                @pl.when(c >= _NBUF)
                def _():
                    out_copy(c - _NBUF, b).wait()

                @plsc.parallel_loop(0, rch * vecs_per_row, unroll=8)
                def inner(i):
                    r = i >> 5
                    col = (i & (vecs_per_row - 1)) * _L
                    idx = idx_bufs[b][r, pl.ds(col, _L)]
                    gidx = (idx << 4) + lanes
                    res_bufs[b][r, pl.ds(col, _L)] = plsc.load_gather(
                        rep_v, [gidx])

                out_copy(c, b).start()

                @pl.when(c + _NBUF < n_chunks)
                def _():
                    in_copy(c + _NBUF, b).start()

            return 0

        lax.fori_loop(0, n_chunks // _NBUF, outer, 0)
        for b0 in range(_NBUF):
            out_copy(n_chunks - _NBUF + b0, b0).wait()

    return k(embT_flat, wq, adj2d)


def kernel(dep_emb_weight, syn_dep_adj, overall_max_len, batch_size, W_q, b_q):
    del overall_max_len, batch_size, b_q
    b, lq, lk = syn_dep_adj.shape
    embT = dep_emb_weight.T.reshape(-1)
    wq = W_q.reshape(-1)
    adj2d = syn_dep_adj.reshape(b * lq, lk)
    out = _dep_type_sc(embT, wq, adj2d)
    return out.reshape(b, lq, lk)


# speedup vs baseline: 1.3584x; 1.0002x over previous
import functools

import jax
import jax.numpy as jnp
from jax import lax
from jax.experimental import pallas as pl
from jax.experimental.pallas import tpu as pltpu
from jax.experimental.pallas import tpu_sc as plsc

_L = 16
_NUM_TYPES = 64
_ATT_DIM = 256
_KB = _NUM_TYPES // _L


_ROWS_PER_CHUNK = 8
_NBUF = 8


@functools.partial(jax.jit, static_argnames=())
def _dep_type_sc(embT_flat, wq, adj2d):
    info = plsc.get_sparse_core_info()
    nc, ns = info.num_cores, info.num_subcores
    nw = nc * ns
    nrows, ncols = adj2d.shape
    assert ncols % _L == 0 and nrows % nw == 0
    rows_per_w = nrows // nw
    rch = _ROWS_PER_CHUNK
    vecs_per_row = ncols // _L
    assert rows_per_w % rch == 0
    n_chunks = rows_per_w // rch
    assert n_chunks % _NBUF == 0 and n_chunks >= 2 * _NBUF

    mesh = plsc.VectorSubcoreMesh(core_axis_name="c", subcore_axis_name="s")

    @functools.partial(
        pl.kernel,
        mesh=mesh,
        out_type=jax.ShapeDtypeStruct((nrows, ncols), jnp.float32),
        compiler_params=pltpu.CompilerParams(needs_layout_passes=False),
        scratch_types=[
            pltpu.VMEM((_ATT_DIM * _NUM_TYPES,), jnp.float32),
            pltpu.VMEM((_ATT_DIM,), jnp.float32),
            pltpu.VMEM((_NUM_TYPES,), jnp.float32),
            pltpu.VMEM((_NUM_TYPES * _L,), jnp.float32),
            *[pltpu.VMEM((rch, ncols), jnp.int32) for _ in range(_NBUF)],
            *[pltpu.VMEM((rch, ncols), jnp.float32) for _ in range(_NBUF)],
            pltpu.VMEM((_L,), jnp.float32),
            *[pltpu.SemaphoreType.DMA for _ in range(2 * _NBUF)],
        ],
    )
    def k(embT_hbm, wq_hbm, adj_hbm, out_hbm, embT_v, wq_v, table_v, rep_v,
          *bufs_and_sems):
        idx_bufs = bufs_and_sems[:_NBUF]
        res_bufs = bufs_and_sems[_NBUF:2 * _NBUF]
        red_v = bufs_and_sems[2 * _NBUF]
        in_sems = bufs_and_sems[2 * _NBUF + 1:2 * _NBUF + 1 + _NBUF]
        out_sems = bufs_and_sems[2 * _NBUF + 1 + _NBUF:]
        wid = lax.axis_index("s") * nc + lax.axis_index("c")
        base = wid * rows_per_w

        def in_copy(c, b):
            return pltpu.make_async_copy(
                adj_hbm.at[pl.ds(base + c * rch, rch)],
                idx_bufs[b], in_sems[b])

        def out_copy(c, b):
            return pltpu.make_async_copy(
                res_bufs[b],
                out_hbm.at[pl.ds(base + c * rch, rch)], out_sems[b])

        for b0 in range(_NBUF):
            in_copy(b0, b0).start()

        pltpu.sync_copy(embT_hbm, embT_v)
        pltpu.sync_copy(wq_hbm, wq_v)

        def matvec_body(db, accs):
            wv = wq_v[pl.ds(db * _L, _L)]
            accs = list(accs)
            for j in range(_L):
                d = db * _L + j
                w = wv[j]
                for kb in range(_KB):
                    accs[kb] = accs[kb] + embT_v[pl.ds(d * _NUM_TYPES + kb * _L, _L)] * w
            return tuple(accs)

        zero = jnp.zeros((_L,), jnp.float32)
        q = lax.fori_loop(0, _ATT_DIM // _L, matvec_body, (zero,) * _KB)

        lanes = lax.iota(jnp.int32, _L)

        def xlane_reduce(v, op):
            for k_sh in (8, 4, 2, 1):
                red_v[pl.ds(0, _L)] = v
                v = op(v, plsc.load_gather(red_v, [lanes ^ k_sh]))
            return v

        m = xlane_reduce(
            jnp.maximum(jnp.maximum(q[0], q[1]), jnp.maximum(q[2], q[3])),
            jnp.maximum,
        )
        e = tuple(jnp.exp(q[kb] - m) for kb in range(_KB))
        s = xlane_reduce(e[0] + e[1] + e[2] + e[3], jnp.add)
        t0 = jnp.where(lanes == 0, jnp.float32(0.0), e[0] / s)
        table_v[pl.ds(0, _L)] = t0
        for kb in range(1, _KB):
            table_v[pl.ds(kb * _L, _L)] = e[kb] / s

        def rep_body(j, _):
            jv = lax.broadcast_in_dim(j, (_L,), ())
            rep_v[pl.ds(j * _L, _L)] = plsc.load_gather(table_v, [jv])
            return 0

        lax.fori_loop(0, _NUM_TYPES, rep_body, 0)

        def outer(grp, _):
            for b in range(_NBUF):
                c = grp * _NBUF + b
                in_copy(c, b).wait()

                @pl.when(c >= _NBUF)
                def _():
                    out_copy(c - _NBUF, b).wait()

                @plsc.parallel_loop(0, rch * vecs_per_row, unroll=8)
                def inner(i):
                    r = i >> 5
                    col = (i & (vecs_per_row - 1)) * _L
                    idx = idx_bufs[b][r, pl.ds(col, _L)]
                    gidx = (idx << 4) + lanes
                    res_bufs[b][r, pl.ds(col, _L)] = plsc.load_gather(
                        rep_v, [gidx])

                out_copy(c, b).start()

                @pl.when(c + _NBUF < n_chunks)
                def _():
                    in_copy(c + _NBUF, b).start()

            return 0

        lax.fori_loop(0, n_chunks // _NBUF, outer, 0)
        for b0 in range(_NBUF):
            out_copy(n_chunks - _NBUF + b0, b0).wait()

    return k(embT_flat, wq, adj2d)


def kernel(dep_emb_weight, syn_dep_adj, overall_max_len, batch_size, W_q, b_q):
    del overall_max_len, batch_size, b_q
    b, lq, lk = syn_dep_adj.shape
    embT = dep_emb_weight.T.reshape(-1)
    wq = W_q.reshape(-1)
    adj2d = syn_dep_adj.reshape(b * lq, lk)
    out = _dep_type_sc(embT, wq, adj2d)
    return out.reshape(b, lq, lk)
